# baseline (speedup 1.0000x reference)
import functools

import jax
import jax.numpy as jnp
from jax import lax
from jax.experimental import pallas as pl
from jax.experimental.pallas import tpu as pltpu
from jax.experimental.pallas import tpu_sc as plsc

_NEMB = 8192
_D = 64
_N = 9216
_TILE = 512
_GRID = _N // _TILE
_SUB = 128
_NCOL = _NEMB // 128

_NW = 32
_BPW = _N // _NW
_CHUNK = 96


def _argmin_body(x_ref, xb_ref, wbT_ref, wT_ref, idx_ref, b2_ref, bmax_ref):
    @pl.when(pl.program_id(0) == 0)
    def _():
        wT = wT_ref[...]
        b2 = jnp.sum(wT * wT, axis=0)
        b2_ref[...] = b2
        bmax_ref[0] = jnp.max(b2)

    x = x_ref[...]
    a2 = jnp.sum(x * x, axis=1, keepdims=True)
    c = jnp.dot(xb_ref[...], wbT_ref[...],
                preferred_element_type=jnp.float32)

    lane = lax.broadcasted_iota(jnp.int32, (_SUB, 128), 1).astype(jnp.float32)

    def tournament(with_b2):
        for r in range(_TILE // _SUB):
            a2r = a2[r * _SUB:(r + 1) * _SUB]

            def chunk_dist(k, r=r, a2r=a2r):
                ck = c[r * _SUB:(r + 1) * _SUB, k * 128:(k + 1) * 128]
                if with_b2:
                    b2k = b2_ref[pl.ds(k * 128, 128)][None, :]
                    return (a2r + b2k) - ck
                return a2r - ck

            run_v = chunk_dist(0)
            run_a = jnp.zeros((_SUB, 128), jnp.float32)
            for k in range(1, _NCOL):
                d = chunk_dist(k)
                upd = d < run_v
                run_v = jnp.where(upd, d, run_v)
                run_a = jnp.where(upd, float(k), run_a)

            m = jnp.min(run_v, axis=1, keepdims=True)
            jf = run_a * 128.0 + lane
            idxf = jnp.min(jnp.where(run_v == m, jf, float(_NEMB)), axis=1)
            idx_ref[pl.ds(r * _SUB, _SUB)] = idxf.astype(jnp.int32)

    collapse = jnp.all((a2 + bmax_ref[0]) == a2)

    @pl.when(collapse)
    def _():
        tournament(with_b2=False)

    @pl.when(jnp.logical_not(collapse))
    def _():
        tournament(with_b2=True)


def _loss_body(q_ref, x_ref, out_ref):
    d = q_ref[...] - x_ref[...]
    v = jnp.sum(d * d) / float(_N * _D)
    out_ref[0, 0] = v + 0.25 * v


@functools.cache
def _make_sc_gather():
    mesh = plsc.VectorSubcoreMesh(core_axis_name="c", subcore_axis_name="s")

    @functools.partial(
        pl.kernel, mesh=mesh,
        out_type=jax.ShapeDtypeStruct((_N, 128), jnp.float32),
        scratch_types=[
            pltpu.VMEM((_BPW,), jnp.int32),
            pltpu.VMEM((_BPW, 128), jnp.float32),
            pltpu.SemaphoreType.DMA,
        ],
    )
    def gather(table_hbm, idx_hbm, out_hbm, idx_v, rows_v, sem):
        wid = lax.axis_index("s") * 2 + lax.axis_index("c")
        base = wid * _BPW
        pltpu.sync_copy(idx_hbm.at[pl.ds(base, _BPW)], idx_v)
        copies = []
        for j in range(_BPW // _CHUNK):
            copies.append(pltpu.async_copy(
                table_hbm.at[idx_v.at[pl.ds(j * _CHUNK, _CHUNK)]],
                rows_v.at[pl.ds(j * _CHUNK, _CHUNK)], sem))
        for cp in copies:
            cp.wait()
        pltpu.sync_copy(rows_v, out_hbm.at[pl.ds(base, _BPW)])

    return gather


def kernel(input, weight):
    x = input.reshape(_N, _D)

    xb = x.astype(jnp.bfloat16)
    wT = weight.T
    wbT = wT.astype(jnp.bfloat16)
    indices = pl.pallas_call(
        _argmin_body,
        grid=(_GRID,),
        in_specs=[
            pl.BlockSpec((_TILE, _D), lambda i: (i, 0)),
            pl.BlockSpec((_TILE, _D), lambda i: (i, 0)),
            pl.BlockSpec((_D, _NEMB), lambda i: (0, 0)),
            pl.BlockSpec((_D, _NEMB), lambda i: (0, 0)),
        ],
        out_specs=pl.BlockSpec((_TILE,), lambda i: (i,)),
        out_shape=jax.ShapeDtypeStruct((_N,), jnp.int32),
        scratch_shapes=[
            pltpu.VMEM((_NEMB,), jnp.float32),
            pltpu.SMEM((1,), jnp.float32),
        ],
    )(x, xb, wbT, wT)

    wpad = jnp.pad(weight, ((0, 0), (0, 128 - _D)))
    qpad = _make_sc_gather()(wpad, indices)
    quantized = qpad[:, :_D]

    loss = pl.pallas_call(
        _loss_body,
        out_specs=pl.BlockSpec(memory_space=pltpu.SMEM),
        out_shape=jax.ShapeDtypeStruct((1, 1), jnp.float32),
    )(quantized, x)[0, 0]

    return quantized.reshape(input.shape), loss


# speedup vs baseline: 1.2366x; 1.2366x over previous
import functools

import jax
import jax.numpy as jnp
from jax import lax
from jax.experimental import pallas as pl
from jax.experimental.pallas import tpu as pltpu
from jax.experimental.pallas import tpu_sc as plsc

_NEMB = 8192
_D = 64
_N = 9216
_TILE = 1024
_GRID = _N // _TILE
_SUB = 128
_NCOL = _NEMB // 128

_NW = 32
_BPW = _N // _NW
_CHUNK = 96


def _argmin_body(x_ref, xb_ref, wbT_ref, wT_ref, idx_ref, b2_ref):
    @pl.when(pl.program_id(0) == 0)
    def _():
        wT = wT_ref[...]
        b2_ref[...] = jnp.sum(wT * wT, axis=0)

    x = x_ref[...]
    a2 = jnp.sum(x * x, axis=1, keepdims=True)
    c = jnp.dot(xb_ref[...], wbT_ref[...],
                preferred_element_type=jnp.float32)

    lane = lax.broadcasted_iota(jnp.int32, (_SUB, 128), 1).astype(jnp.float32)

    for r in range(_TILE // _SUB):
        a2r = a2[r * _SUB:(r + 1) * _SUB]

        def chunk_dist(k, r=r, a2r=a2r):
            b2k = b2_ref[pl.ds(k * 128, 128)][None, :]
            ck = c[r * _SUB:(r + 1) * _SUB, k * 128:(k + 1) * 128]
            return (a2r + b2k) - ck

        run_v = chunk_dist(0)
        run_a = jnp.zeros((_SUB, 128), jnp.float32)
        for k in range(1, _NCOL):
            d = chunk_dist(k)
            upd = d < run_v
            run_v = jnp.where(upd, d, run_v)
            run_a = jnp.where(upd, float(k), run_a)

        m = jnp.min(run_v, axis=1, keepdims=True)
        jf = run_a * 128.0 + lane
        idxf = jnp.min(jnp.where(run_v == m, jf, float(_NEMB)), axis=1)
        idx_ref[pl.ds(r * _SUB, _SUB)] = idxf.astype(jnp.int32)


def _loss_body(q_ref, x_ref, out_ref):
    d = q_ref[...] - x_ref[...]
    v = jnp.sum(d * d) / float(_N * _D)
    out_ref[0, 0] = v + 0.25 * v


@functools.cache
def _make_sc_gather():
    mesh = plsc.VectorSubcoreMesh(core_axis_name="c", subcore_axis_name="s")

    @functools.partial(
        pl.kernel, mesh=mesh,
        out_type=jax.ShapeDtypeStruct((_N, 128), jnp.float32),
        scratch_types=[
            pltpu.VMEM((_BPW,), jnp.int32),
            pltpu.VMEM((_BPW, 128), jnp.float32),
            pltpu.SemaphoreType.DMA,
        ],
    )
    def gather(table_hbm, idx_hbm, out_hbm, idx_v, rows_v, sem):
        wid = lax.axis_index("s") * 2 + lax.axis_index("c")
        base = wid * _BPW
        pltpu.sync_copy(idx_hbm.at[pl.ds(base, _BPW)], idx_v)
        copies = []
        for j in range(_BPW // _CHUNK):
            copies.append(pltpu.async_copy(
                table_hbm.at[idx_v.at[pl.ds(j * _CHUNK, _CHUNK)]],
                rows_v.at[pl.ds(j * _CHUNK, _CHUNK)], sem))
        for cp in copies:
            cp.wait()
        pltpu.sync_copy(rows_v, out_hbm.at[pl.ds(base, _BPW)])

    return gather


def kernel(input, weight):
    x = input.reshape(_N, _D)

    xb = x.astype(jnp.bfloat16)
    wT = weight.T
    wbT = wT.astype(jnp.bfloat16)
    indices = pl.pallas_call(
        _argmin_body,
        grid=(_GRID,),
        in_specs=[
            pl.BlockSpec((_TILE, _D), lambda i: (i, 0)),
            pl.BlockSpec((_TILE, _D), lambda i: (i, 0)),
            pl.BlockSpec((_D, _NEMB), lambda i: (0, 0)),
            pl.BlockSpec((_D, _NEMB), lambda i: (0, 0)),
        ],
        out_specs=pl.BlockSpec((_TILE,), lambda i: (i,)),
        out_shape=jax.ShapeDtypeStruct((_N,), jnp.int32),
        scratch_shapes=[
            pltpu.VMEM((_NEMB,), jnp.float32),
        ],
    )(x, xb, wbT, wT)

    wpad = jnp.pad(weight, ((0, 0), (0, 128 - _D)))
    qpad = _make_sc_gather()(wpad, indices)
    quantized = qpad[:, :_D]

    loss = pl.pallas_call(
        _loss_body,
        out_specs=pl.BlockSpec(memory_space=pltpu.SMEM),
        out_shape=jax.ShapeDtypeStruct((1, 1), jnp.float32),
    )(quantized, x)[0, 0]

    return quantized.reshape(input.shape), loss
